# baseline (speedup 1.0000x reference)
import jax
import jax.numpy as jnp
from jax import lax
from jax.experimental import pallas as pl
from jax.experimental.pallas import tpu as pltpu
from jax.experimental.pallas import tpu_sc as plsc

BATCH = 4096
SEQ = 200
EMBED = 64
LANES = 16

NUM_CORES = 2
NUM_SUBCORES = 16
NW = NUM_CORES * NUM_SUBCORES
BATCH_PER_W = BATCH // NW
CB = 2
CROWS = CB * SEQ
NCHUNK = BATCH_PER_W // CB
ROWS_PER_W = BATCH_PER_W * SEQ
NB = 3


def _sc_body(x_hbm, pidx_hbm, tab_hbm, pos_hbm, out_hbm, idx_v, pidx_v,
             pshared, buf_v, sem_g, sem_p, sem_o):
    wid = lax.axis_index("s") * NUM_CORES + lax.axis_index("c")
    row0 = wid * ROWS_PER_W

    pltpu.sync_copy(x_hbm.at[pl.ds(row0, ROWS_PER_W)], idx_v)
    pltpu.sync_copy(pidx_hbm, pidx_v)
    @pl.when(lax.axis_index("s") == 0)
    def _():
        pltpu.sync_copy(pos_hbm, pshared)
    plsc.subcore_barrier()

    def g_start(j, slot):
        pltpu.async_copy(tab_hbm.at[idx_v.at[pl.ds(CROWS * j, CROWS)]],
                         buf_v.at[slot], sem_g.at[slot])

    def g_wait(slot):
        pltpu.make_async_copy(tab_hbm.at[idx_v.at[pl.ds(0, CROWS)]],
                              buf_v.at[slot], sem_g.at[slot]).wait()

    def o_start(j, slot):
        pltpu.async_copy(buf_v.at[slot],
                         out_hbm.at[pl.ds(row0 + CROWS * j, CROWS)],
                         sem_o.at[slot])

    def o_wait(j, slot):
        pltpu.make_async_copy(buf_v.at[slot],
                              out_hbm.at[pl.ds(row0 + CROWS * j, CROWS)],
                              sem_o.at[slot]).wait()

    def add_pos(slot):
        pltpu.async_copy(pshared.at[pidx_v], buf_v.at[slot], sem_p.at[0],
                         add=True)
        pltpu.make_async_copy(pshared.at[pidx_v], buf_v.at[slot],
                              sem_p.at[0]).wait()

    g_start(0, 0)
    g_start(1, 1)
    g_wait(0)
    add_pos(0)
    o_start(0, 0)
    g_start(2, 2)

    def body(j, carry):
        slot0 = lax.rem(j, NB)
        slot2 = lax.rem(j + 2, NB)
        g_wait(slot0)
        add_pos(slot0)
        o_start(j, slot0)
        o_wait(j - 1, slot2)
        g_start(j + 2, slot2)
        return carry

    lax.fori_loop(1, NCHUNK - 2, body, 0)

    j = NCHUNK - 2
    g_wait(j % NB)
    add_pos(j % NB)
    o_start(j, j % NB)
    o_wait(j - 1, (j + 2) % NB)
    j = NCHUNK - 1
    g_wait(j % NB)
    add_pos(j % NB)
    o_start(j, j % NB)
    o_wait(NCHUNK - 2, (NCHUNK - 2) % NB)
    o_wait(NCHUNK - 1, (NCHUNK - 1) % NB)


@jax.jit
def kernel(x, token_table, pos_table):
    x_flat = x.reshape(BATCH * SEQ)
    pos_idx = jnp.tile(jnp.arange(SEQ, dtype=jnp.int32), CB)
    mesh = plsc.VectorSubcoreMesh(core_axis_name="c", subcore_axis_name="s")
    f = pl.kernel(
        _sc_body,
        out_type=jax.ShapeDtypeStruct((BATCH * SEQ, EMBED), jnp.float32),
        mesh=mesh,
        compiler_params=pltpu.CompilerParams(use_tc_tiling_on_sc=False),
        scratch_types=[
            pltpu.VMEM((ROWS_PER_W,), jnp.int32),
            pltpu.VMEM((CROWS,), jnp.int32),
            pltpu.VMEM_SHARED((SEQ, EMBED), jnp.float32),
            pltpu.VMEM((NB, CROWS, EMBED), jnp.float32),
            pltpu.SemaphoreType.DMA((NB,)),
            pltpu.SemaphoreType.DMA((1,)),
            pltpu.SemaphoreType.DMA((NB,)),
        ],
    )
    out = f(x_flat, pos_idx, token_table, pos_table)
    return out.reshape(BATCH, SEQ, EMBED)
